# Initial kernel scaffold; baseline (speedup 1.0000x reference)
import jax
import jax.numpy as jnp
from jax.experimental import pallas as pl


def kernel(token_ids, emb_0, emb_1, emb_2, proj_0, proj_1, proj_2):
    raise NotImplementedError("write your pallas kernel here")


# speedup vs baseline: 2.5332x; 2.5332x over previous
import functools

import jax
import jax.numpy as jnp
from jax import lax
from jax.experimental import pallas as pl
from jax.experimental.pallas import tpu as pltpu
from jax.experimental.pallas import tpu_sc as plsc

VOCAB_ = 100000
EMB_ = 512
HID_ = 512
ENDS_ = (0, 20000, 60000, 100000)
ROWS_PER_BLOCK = 800
SCALE_ = float(HID_) ** 0.5


def _table_body(emb0, emb1, emb2, p0, p1, p2, out):
    g = pl.program_id(0)
    nb0 = (ENDS_[1] - ENDS_[0]) // ROWS_PER_BLOCK
    nb1 = (ENDS_[2] - ENDS_[1]) // ROWS_PER_BLOCK

    def proj(eref, pref):
        return lax.dot_general(
            eref[...], pref[...], (((1,), (1,)), ((), ())),
            preferred_element_type=jnp.float32,
        ) * SCALE_

    @pl.when(g < nb0)
    def _():
        out[...] = proj(emb0, p0)

    @pl.when((g >= nb0) & (g < nb0 + nb1))
    def _():
        out[...] = proj(emb1, p1)

    @pl.when(g >= nb0 + nb1)
    def _():
        out[...] = proj(emb2, p2)


def _build_table(emb_0, emb_1, emb_2, proj_0, proj_1, proj_2):
    r = ROWS_PER_BLOCK
    nb0 = (ENDS_[1] - ENDS_[0]) // r
    nb1 = (ENDS_[2] - ENDS_[1]) // r
    nb2 = (ENDS_[3] - ENDS_[2]) // r
    grid = nb0 + nb1 + nb2
    return pl.pallas_call(
        _table_body,
        grid=(grid,),
        in_specs=[
            pl.BlockSpec((r, EMB_), lambda g: (jnp.minimum(g, nb0 - 1), 0)),
            pl.BlockSpec((r, EMB_ // 2),
                         lambda g: (jnp.clip(g - nb0, 0, nb1 - 1), 0)),
            pl.BlockSpec((r, EMB_ // 4),
                         lambda g: (jnp.clip(g - nb0 - nb1, 0, nb2 - 1), 0)),
            pl.BlockSpec((HID_, EMB_), lambda g: (0, 0)),
            pl.BlockSpec((HID_, EMB_ // 2), lambda g: (0, 0)),
            pl.BlockSpec((HID_, EMB_ // 4), lambda g: (0, 0)),
        ],
        out_specs=pl.BlockSpec((r, HID_), lambda g: (g, 0)),
        out_shape=jax.ShapeDtypeStruct((VOCAB_, HID_), jnp.float32),
    )(emb_0, emb_1, emb_2, proj_0, proj_1, proj_2)


@functools.cache
def _make_gather(b_total):
    info = plsc.get_sparse_core_info()
    nc, ns = info.num_cores, info.num_subcores
    nw = nc * ns
    assert b_total % nw == 0
    b_per_w = b_total // nw
    chunk = 80
    assert b_per_w % chunk == 0
    n_chunks = b_per_w // chunk
    mesh = plsc.VectorSubcoreMesh(core_axis_name="c", subcore_axis_name="s")

    @functools.partial(
        pl.kernel,
        mesh=mesh,
        out_type=jax.ShapeDtypeStruct((b_total, HID_), jnp.float32),
        scratch_types=[
            pltpu.VMEM((b_per_w,), jnp.int32),
            pltpu.VMEM((chunk, HID_), jnp.float32),
            pltpu.VMEM((chunk, HID_), jnp.float32),
            pltpu.SemaphoreType.DMA,
            pltpu.SemaphoreType.DMA,
        ],
    )
    def gather(table_hbm, idx_hbm, out_hbm, idx_v, rows_a, rows_b, sem_a,
               sem_b):
        wid = lax.axis_index("s") * nc + lax.axis_index("c")
        base = wid * b_per_w
        pltpu.sync_copy(idx_hbm.at[pl.ds(base, b_per_w)], idx_v)
        bufs = (rows_a, rows_b)
        sems = (sem_a, sem_b)
        copies = [None, None]
        copies[0] = pltpu.async_copy(
            table_hbm.at[idx_v.at[pl.ds(0, chunk)]], bufs[0], sems[0])
        for c in range(n_chunks):
            if c + 1 < n_chunks:
                copies[(c + 1) % 2] = pltpu.async_copy(
                    table_hbm.at[idx_v.at[pl.ds((c + 1) * chunk, chunk)]],
                    bufs[(c + 1) % 2], sems[(c + 1) % 2])
            copies[c % 2].wait()
            pltpu.sync_copy(bufs[c % 2],
                            out_hbm.at[pl.ds(base + c * chunk, chunk)])

    return gather


def kernel(token_ids, emb_0, emb_1, emb_2, proj_0, proj_1, proj_2):
    table = _build_table(emb_0, emb_1, emb_2, proj_0, proj_1, proj_2)
    flat = token_ids.reshape(-1).astype(jnp.int32)
    out = _make_gather(flat.shape[0])(table, flat)
    return out.reshape(token_ids.shape + (HID_,))
